# baseline (speedup 1.0000x reference)
import math

import jax
import jax.numpy as jnp
from jax.experimental import pallas as pl

B = 16384
SD = 128
AD = 32
TD = 16
MD = 128
E = 16
K = 4
ID = SD + AD + TD

TILE = 4096


def _mish(v):
    u = jnp.exp(jnp.minimum(v, 30.0))
    num = u * (u + 2.0)
    return v * (num / (num + 2.0))


def _fused_kernel(x_ref, time_ref, state_ref,
                  tW1_ref, tb1_ref, tW2_ref, tb2_ref,
                  gW1_ref, gb1_ref, gW2_ref, gb2_ref,
                  eW1_ref, eb1_ref, eW2_ref, eb2_ref, eW3_ref, eb3_ref,
                  fW_ref, fb_ref,
                  out_ref, cnt_ref, ent_ref):
    f32 = jnp.float32
    xv = x_ref[...]
    sv = state_ref[...]
    tv = time_ref[...]

    half = TD // 2
    lane = jax.lax.broadcasted_iota(jnp.int32, (1, half), 1).astype(f32)
    freq = jnp.exp(lane * (-math.log(10000.0) / (half - 1)))
    emb = tv * freq
    y = emb * emb
    se = emb * (1.0 + y * (-1.0 / 6.0 + y * (1.0 / 120.0 - y * (1.0 / 5040.0))))
    ce = 1.0 + y * (-0.5 + y * (1.0 / 24.0 + y * (-1.0 / 720.0 + y * (1.0 / 40320.0))))

    t1 = se @ tW1_ref[0:half, :] + ce @ tW1_ref[half:TD, :] + tb1_ref[0:1, :]
    t1 = _mish(t1)
    tvec = t1 @ tW2_ref[...] + tb2_ref[0:1, :]

    g1 = (xv @ gW1_ref[0:AD, :]
          + tvec @ gW1_ref[AD:AD + TD, :]
          + sv @ gW1_ref[AD + TD:ID, :]
          + gb1_ref[0:1, :])
    g1 = jnp.maximum(g1, 0.0)
    logits = g1 @ gW2_ref[...] + gb2_ref[0:1, :]

    m = jnp.max(logits, axis=1, keepdims=True)
    ex = jnp.exp(logits - m)
    z = jnp.sum(ex, axis=1, keepdims=True)
    p = ex / z

    eidx = jax.lax.broadcasted_iota(jnp.int32, p.shape, 1)
    work = p
    sel = jnp.zeros_like(p)
    for _ in range(K):
        mk = jnp.max(work, axis=1, keepdims=True)
        cand = jnp.where(work == mk, eidx, E)
        amin = jnp.min(cand, axis=1, keepdims=True)
        first = (eidx == amin)
        sel = sel + jnp.where(first, 1.0, 0.0)
        work = jnp.where(first, -1.0, work)
    psel = p * sel
    tsum = jnp.sum(psel, axis=1, keepdims=True)
    wnorm = psel / (tsum + 1e-9)

    @pl.when(pl.program_id(0) == 0)
    def _init():
        cnt_ref[...] = jnp.zeros_like(cnt_ref)
        ent_ref[...] = jnp.zeros_like(ent_ref)

    cnt_ref[...] += jnp.sum(sel, axis=0, keepdims=True)
    ent_row = jnp.log(z) - jnp.sum(p * (logits - m), axis=1, keepdims=True)
    ent_ref[...] += jnp.sum(ent_row).reshape(1, 1)

    acc = jnp.zeros((xv.shape[0], MD), f32)
    for e in range(E):
        h1 = (xv @ eW1_ref[e, 0:AD, :]
              + tvec @ eW1_ref[e, AD:AD + TD, :]
              + sv @ eW1_ref[e, AD + TD:ID, :]
              + eb1_ref[e:e + 1, :])
        h1 = _mish(h1)
        h2 = _mish(h1 @ eW2_ref[e] + eb2_ref[e:e + 1, :])
        h3 = _mish(h2 @ eW3_ref[e] + eb3_ref[e:e + 1, :])
        acc = acc + wnorm[:, e:e + 1] * h3

    out_ref[...] = acc @ fW_ref[...] + fb_ref[0:1, :]


def kernel(x, time, state, tW1, tb1, tW2, tb2, gW1, gb1, gW2, gb2,
           eW1, eb1, eW2, eb2, eW3, eb3, fW, fb):
    time2 = time.reshape(B, 1)
    tb1r = tb1.reshape(1, -1)
    tb2r = tb2.reshape(1, -1)
    gb1r = gb1.reshape(1, -1)
    gb2r = gb2.reshape(1, -1)
    fbr = fb.reshape(1, -1)

    grid = (B // TILE,)

    def row_blk(cols):
        return pl.BlockSpec((TILE, cols), lambda i: (i, 0))

    def full2(a):
        return pl.BlockSpec(a.shape, lambda i: (0,) * a.ndim)

    out, cnt, ent = pl.pallas_call(
        _fused_kernel,
        grid=grid,
        in_specs=[
            row_blk(AD),
            row_blk(1),
            row_blk(SD),
            full2(tW1), full2(tb1r), full2(tW2), full2(tb2r),
            full2(gW1), full2(gb1r), full2(gW2), full2(gb2r),
            full2(eW1), full2(eb1), full2(eW2), full2(eb2),
            full2(eW3), full2(eb3),
            full2(fW), full2(fbr),
        ],
        out_specs=[
            pl.BlockSpec((TILE, AD), lambda i: (i, 0)),
            pl.BlockSpec((1, E), lambda i: (0, 0)),
            pl.BlockSpec((1, 1), lambda i: (0, 0)),
        ],
        out_shape=[
            jax.ShapeDtypeStruct((B, AD), jnp.float32),
            jax.ShapeDtypeStruct((1, E), jnp.float32),
            jax.ShapeDtypeStruct((1, 1), jnp.float32),
        ],
    )(x, time2, state, tW1, tb1r, tW2, tb2r, gW1, gb1r, gW2, gb2r,
      eW1, eb1, eW2, eb2, eW3, eb3, fW, fbr)

    expert_load = cnt[0] / (B + 1e-9)
    load_balancing_loss = jnp.var(expert_load, ddof=1)
    entropy = ent[0, 0] / B
    aux_loss = load_balancing_loss + entropy
    return (out, aux_loss)


# speedup vs baseline: 1.2320x; 1.2320x over previous
import math

import jax
import jax.numpy as jnp
from jax.experimental import pallas as pl

B = 16384
SD = 128
AD = 32
TD = 16
MD = 128
E = 16
K = 4
ID = SD + AD + TD

TILE = 2048


def _mish(v):
    u = jnp.exp(jnp.minimum(v, 30.0))
    num = u * (u + 2.0)
    return v * (num / (num + 2.0))


def _fused_kernel(x_ref, time_ref, state_ref,
                  tW1_ref, tb1_ref, tW2_ref, tb2_ref,
                  gW1_ref, gb1_ref, gW2_ref, gb2_ref,
                  eW1_ref, eb1_ref, eW2_ref, eb2_ref, eW3_ref, eb3_ref,
                  fW_ref, fb_ref,
                  out_ref, cnt_ref, ent_ref):
    f32 = jnp.float32
    xv = x_ref[...]
    sv = state_ref[...]
    tv = time_ref[...]

    half = TD // 2
    lane = jax.lax.broadcasted_iota(jnp.int32, (1, half), 1).astype(f32)
    freq = jnp.exp(lane * (-math.log(10000.0) / (half - 1)))
    emb = tv * freq
    y = emb * emb
    se = emb * (1.0 + y * (-1.0 / 6.0 + y * (1.0 / 120.0 - y * (1.0 / 5040.0))))
    ce = 1.0 + y * (-0.5 + y * (1.0 / 24.0 + y * (-1.0 / 720.0 + y * (1.0 / 40320.0))))

    t1 = se @ tW1_ref[0:half, :] + ce @ tW1_ref[half:TD, :] + tb1_ref[0:1, :]
    t1 = _mish(t1)
    tvec = t1 @ tW2_ref[...] + tb2_ref[0:1, :]

    g1 = (xv @ gW1_ref[0:AD, :]
          + tvec @ gW1_ref[AD:AD + TD, :]
          + sv @ gW1_ref[AD + TD:ID, :]
          + gb1_ref[0:1, :])
    g1 = jnp.maximum(g1, 0.0)
    logits = g1 @ gW2_ref[...] + gb2_ref[0:1, :]

    m = jnp.max(logits, axis=1, keepdims=True)
    ex = jnp.exp(logits - m)
    z = jnp.sum(ex, axis=1, keepdims=True)
    p = ex / z

    eidx = jax.lax.broadcasted_iota(jnp.int32, p.shape, 1)
    work = p
    sel = jnp.zeros_like(p)
    for _ in range(K):
        mk = jnp.max(work, axis=1, keepdims=True)
        cand = jnp.where(work == mk, eidx, E)
        amin = jnp.min(cand, axis=1, keepdims=True)
        first = (eidx == amin)
        sel = sel + jnp.where(first, 1.0, 0.0)
        work = jnp.where(first, -1.0, work)
    psel = p * sel
    tsum = jnp.sum(psel, axis=1, keepdims=True)
    wnorm = psel / (tsum + 1e-9)

    @pl.when(pl.program_id(0) == 0)
    def _init():
        cnt_ref[...] = jnp.zeros_like(cnt_ref)
        ent_ref[...] = jnp.zeros_like(ent_ref)

    cnt_ref[...] += jnp.sum(sel, axis=0, keepdims=True)
    ent_row = jnp.log(z) - jnp.sum(p * (logits - m), axis=1, keepdims=True)
    ent_ref[...] += jnp.sum(ent_row).reshape(1, 1)

    acc = jnp.zeros((xv.shape[0], MD), f32)
    for e in range(E):
        h1 = (xv @ eW1_ref[e, 0:AD, :]
              + tvec @ eW1_ref[e, AD:AD + TD, :]
              + sv @ eW1_ref[e, AD + TD:ID, :]
              + eb1_ref[e:e + 1, :])
        h1 = _mish(h1)
        h2 = _mish(h1 @ eW2_ref[e] + eb2_ref[e:e + 1, :])
        h3 = _mish(h2 @ eW3_ref[e] + eb3_ref[e:e + 1, :])
        acc = acc + wnorm[:, e:e + 1] * h3

    out_ref[...] = acc @ fW_ref[...] + fb_ref[0:1, :]


def kernel(x, time, state, tW1, tb1, tW2, tb2, gW1, gb1, gW2, gb2,
           eW1, eb1, eW2, eb2, eW3, eb3, fW, fb):
    time2 = time.reshape(B, 1)
    tb1r = tb1.reshape(1, -1)
    tb2r = tb2.reshape(1, -1)
    gb1r = gb1.reshape(1, -1)
    gb2r = gb2.reshape(1, -1)
    fbr = fb.reshape(1, -1)

    grid = (B // TILE,)

    def row_blk(cols):
        return pl.BlockSpec((TILE, cols), lambda i: (i, 0))

    def full2(a):
        return pl.BlockSpec(a.shape, lambda i: (0,) * a.ndim)

    out, cnt, ent = pl.pallas_call(
        _fused_kernel,
        grid=grid,
        in_specs=[
            row_blk(AD),
            row_blk(1),
            row_blk(SD),
            full2(tW1), full2(tb1r), full2(tW2), full2(tb2r),
            full2(gW1), full2(gb1r), full2(gW2), full2(gb2r),
            full2(eW1), full2(eb1), full2(eW2), full2(eb2),
            full2(eW3), full2(eb3),
            full2(fW), full2(fbr),
        ],
        out_specs=[
            pl.BlockSpec((TILE, AD), lambda i: (i, 0)),
            pl.BlockSpec((1, E), lambda i: (0, 0)),
            pl.BlockSpec((1, 1), lambda i: (0, 0)),
        ],
        out_shape=[
            jax.ShapeDtypeStruct((B, AD), jnp.float32),
            jax.ShapeDtypeStruct((1, E), jnp.float32),
            jax.ShapeDtypeStruct((1, 1), jnp.float32),
        ],
    )(x, time2, state, tW1, tb1r, tW2, tb2r, gW1, gb1r, gW2, gb2r,
      eW1, eb1, eW2, eb2, eW3, eb3, fW, fbr)

    expert_load = cnt[0] / (B + 1e-9)
    load_balancing_loss = jnp.var(expert_load, ddof=1)
    entropy = ent[0, 0] / B
    aux_loss = load_balancing_loss + entropy
    return (out, aux_loss)
